# baseline (speedup 1.0000x reference)
import functools

import jax
import jax.numpy as jnp
from jax import lax
from jax.experimental import pallas as pl
from jax.experimental.pallas import tpu as pltpu
from jax.experimental.pallas import tpu_sc as plsc

_N = 100000
_C = 32
_K = 16
_H = 8
_M = 16


def _sc_gather(table, idx):
    info = plsc.get_sparse_core_info()
    nw = info.num_cores * info.num_subcores
    nk = idx.shape[0]
    b_per_w = nk // nw
    chunk = 2000
    n_iter = b_per_w // chunk
    assert b_per_w % chunk == 0 and chunk % 8 == 0 and b_per_w % 8 == 0

    mesh = plsc.VectorSubcoreMesh(core_axis_name="c", subcore_axis_name="s")

    @functools.partial(
        pl.kernel,
        mesh=mesh,
        out_type=jax.ShapeDtypeStruct((nk, _C), jnp.bfloat16),
        scratch_types=[
            pltpu.VMEM((chunk,), jnp.int32),
            pltpu.VMEM((chunk, _C), jnp.bfloat16),
            pltpu.SemaphoreType.DMA,
        ],
        compiler_params=pltpu.CompilerParams(use_tc_tiling_on_sc=False),
    )
    def gather_kernel(table_hbm, idx_hbm, out_hbm, idx_v, rows_v, sem):
        wid = lax.axis_index("s") * info.num_cores + lax.axis_index("c")
        base = wid * b_per_w
        for i in range(n_iter):
            off = base + i * chunk
            pltpu.sync_copy(idx_hbm.at[pl.ds(off, chunk)], idx_v)
            pltpu.async_copy(table_hbm.at[idx_v], rows_v, sem).wait()
            pltpu.sync_copy(rows_v, out_hbm.at[pl.ds(off, chunk)])

    return gather_kernel(table, idx)


def _tc_contract(gathered2, guid2, w2):
    P = 1000
    n = gathered2.shape[0]
    cm = _C * _M

    def body(g_ref, d_ref, w_ref, o_ref):
        q = lax.broadcasted_iota(jnp.int32, (_K * _H, _K * _C), 0)
        j = lax.broadcasted_iota(jnp.int32, (_K * _H, _K * _C), 1)
        e8a = ((q // _H == j // _C) & (q % _H == (j % _C) // 4))
        e8a = e8a.astype(jnp.float32)

        col = lax.broadcasted_iota(jnp.int32, (_C, cm), 1)
        row = lax.broadcasted_iota(jnp.int32, (_C, cm), 0)
        e32 = (col // _M == row).astype(jnp.float32)

        g = g_ref[:].astype(jnp.float32)
        d = d_ref[:]
        w = w_ref[:]
        guided = g * jnp.dot(d, e8a, preferred_element_type=jnp.float32)
        acc = None
        for k0 in range(0, _K, 4):
            terms = []
            for k in range(k0, k0 + 4):
                gk = guided[:, k * _C:(k + 1) * _C]
                wk = w[:, k * _M:(k + 1) * _M]
                grep = jnp.dot(gk, e32, preferred_element_type=jnp.float32)
                wrep = jnp.tile(wk, (1, _C))
                terms.append(grep * wrep)
            part = (terms[0] + terms[1]) + (terms[2] + terms[3])
            acc = part if acc is None else acc + part
        o_ref[:] = acc

    return pl.pallas_call(
        body,
        grid=(n // P,),
        in_specs=[
            pl.BlockSpec((P, _K * _C), lambda i: (i, 0)),
            pl.BlockSpec((P, _K * _H), lambda i: (i, 0)),
            pl.BlockSpec((P, _K * _M), lambda i: (i, 0)),
        ],
        out_specs=pl.BlockSpec((P, cm), lambda i: (i, 0)),
        out_shape=jax.ShapeDtypeStruct((n, cm), jnp.float32),
    )(gathered2, guid2, w2)


def kernel(input_features, neighbor_inds, guidance, weightnet):
    b, n, c = input_features.shape
    k = neighbor_inds.shape[2]
    h = guidance.shape[3]
    m = weightnet.shape[3]

    table = input_features.reshape(n, c).astype(jnp.bfloat16)
    idx = neighbor_inds.reshape(n * k).astype(jnp.int32)
    gathered = _sc_gather(table, idx)

    out = _tc_contract(
        gathered.reshape(n, k * c),
        guidance.reshape(n, k * h),
        weightnet.reshape(n, k * m),
    )
    return out.reshape(b, n, c * m)


# speedup vs baseline: 1.1576x; 1.1576x over previous
import functools

import jax
import jax.numpy as jnp
from jax import lax
from jax.experimental import pallas as pl
from jax.experimental.pallas import tpu as pltpu
from jax.experimental.pallas import tpu_sc as plsc

_N = 100000
_C = 32
_K = 16
_H = 8
_M = 16


def _sc_gather(table, idx):
    info = plsc.get_sparse_core_info()
    nw = info.num_cores * info.num_subcores
    nk = idx.shape[0]
    b_per_w = nk // nw
    chunk = 2000
    n_iter = b_per_w // chunk
    assert b_per_w % chunk == 0 and chunk % 8 == 0 and b_per_w % 8 == 0

    mesh = plsc.VectorSubcoreMesh(core_axis_name="c", subcore_axis_name="s")

    @functools.partial(
        pl.kernel,
        mesh=mesh,
        out_type=jax.ShapeDtypeStruct((nk, _C), jnp.float32),
        scratch_types=[
            pltpu.VMEM((chunk,), jnp.int32),
            pltpu.VMEM((chunk, _C), jnp.float32),
            pltpu.SemaphoreType.DMA,
        ],
        compiler_params=pltpu.CompilerParams(use_tc_tiling_on_sc=False),
    )
    def gather_kernel(table_hbm, idx_hbm, out_hbm, idx_v, rows_v, sem):
        wid = lax.axis_index("s") * info.num_cores + lax.axis_index("c")
        base = wid * b_per_w
        for i in range(n_iter):
            off = base + i * chunk
            pltpu.sync_copy(idx_hbm.at[pl.ds(off, chunk)], idx_v)
            pltpu.async_copy(table_hbm.at[idx_v], rows_v, sem).wait()
            pltpu.sync_copy(rows_v, out_hbm.at[pl.ds(off, chunk)])

    return gather_kernel(table, idx)


def _tc_contract(gathered2, guid2, w2):
    P = 1000
    n = gathered2.shape[0]
    cm = _C * _M

    def body(g_ref, d_ref, w_ref, o_ref):
        q = lax.broadcasted_iota(jnp.int32, (_K * _H, _K * _C), 0)
        j = lax.broadcasted_iota(jnp.int32, (_K * _H, _K * _C), 1)
        e8a = ((q // _H == j // _C) & (q % _H == (j % _C) // 4))
        e8a = e8a.astype(jnp.float32)

        col = lax.broadcasted_iota(jnp.int32, (_C, cm), 1)
        row = lax.broadcasted_iota(jnp.int32, (_C, cm), 0)
        e32 = (col // _M == row).astype(jnp.float32)

        g = g_ref[:]
        d = d_ref[:]
        w = w_ref[:]
        guided = g * jnp.dot(d, e8a, preferred_element_type=jnp.float32)
        acc = None
        for k0 in range(0, _K, 4):
            terms = []
            for k in range(k0, k0 + 4):
                gk = guided[:, k * _C:(k + 1) * _C]
                wk = w[:, k * _M:(k + 1) * _M]
                grep = jnp.dot(gk, e32, preferred_element_type=jnp.float32)
                wrep = jnp.tile(wk, (1, _C))
                terms.append(grep * wrep)
            part = (terms[0] + terms[1]) + (terms[2] + terms[3])
            acc = part if acc is None else acc + part
        o_ref[:] = acc

    return pl.pallas_call(
        body,
        grid=(n // P,),
        in_specs=[
            pl.BlockSpec((P, _K * _C), lambda i: (i, 0)),
            pl.BlockSpec((P, _K * _H), lambda i: (i, 0)),
            pl.BlockSpec((P, _K * _M), lambda i: (i, 0)),
        ],
        out_specs=pl.BlockSpec((P, cm), lambda i: (i, 0)),
        out_shape=jax.ShapeDtypeStruct((n, cm), jnp.float32),
    )(gathered2, guid2, w2)


def kernel(input_features, neighbor_inds, guidance, weightnet):
    b, n, c = input_features.shape
    k = neighbor_inds.shape[2]
    h = guidance.shape[3]
    m = weightnet.shape[3]

    table = input_features.reshape(n, c)
    idx = neighbor_inds.reshape(n * k).astype(jnp.int32)
    gathered = _sc_gather(table, idx)

    out = _tc_contract(
        gathered.reshape(n, k * c),
        guidance.reshape(n, k * h),
        weightnet.reshape(n, k * m),
    )
    return out.reshape(b, n, c * m)


# speedup vs baseline: 1.1626x; 1.0043x over previous
import functools

import jax
import jax.numpy as jnp
from jax import lax
from jax.experimental import pallas as pl
from jax.experimental.pallas import tpu as pltpu
from jax.experimental.pallas import tpu_sc as plsc

_N = 100000
_C = 32
_K = 16
_H = 8
_M = 16


def _sc_gather(table, idx):
    info = plsc.get_sparse_core_info()
    nw = info.num_cores * info.num_subcores
    nk = idx.shape[0]
    b_per_w = nk // nw
    chunk = 1000
    n_iter = b_per_w // chunk
    assert b_per_w % chunk == 0 and chunk % 8 == 0 and b_per_w % 8 == 0

    mesh = plsc.VectorSubcoreMesh(core_axis_name="c", subcore_axis_name="s")

    @functools.partial(
        pl.kernel,
        mesh=mesh,
        out_type=jax.ShapeDtypeStruct((nk, _C), jnp.float32),
        scratch_types=[
            pltpu.VMEM((b_per_w,), jnp.int32),
            pltpu.VMEM((2, chunk, _C), jnp.float32),
            pltpu.SemaphoreType.DMA,
            pltpu.SemaphoreType.DMA,
        ],
        compiler_params=pltpu.CompilerParams(use_tc_tiling_on_sc=False),
    )
    def gather_kernel(table_hbm, idx_hbm, out_hbm, idx_v, rows_v, sem0, sem1):
        wid = lax.axis_index("s") * info.num_cores + lax.axis_index("c")
        base = wid * b_per_w
        pltpu.sync_copy(idx_hbm.at[pl.ds(base, b_per_w)], idx_v)
        sems = (sem0, sem1)

        def start(i):
            return pltpu.async_copy(
                table_hbm.at[idx_v.at[pl.ds(i * chunk, chunk)]],
                rows_v.at[i % 2],
                sems[i % 2],
            )
        copies = [start(0)]
        for i in range(n_iter):
            if i + 1 < n_iter:
                copies.append(start(i + 1))
            copies[i].wait()
            pltpu.sync_copy(rows_v.at[i % 2],
                            out_hbm.at[pl.ds(base + i * chunk, chunk)])

    return gather_kernel(table, idx)


def _tc_contract(gathered2, guid2, w2):
    P = 1000
    n = gathered2.shape[0]
    cm = _C * _M

    def body(g_ref, d_ref, w_ref, o_ref):
        q = lax.broadcasted_iota(jnp.int32, (_K * _H, _K * _C), 0)
        j = lax.broadcasted_iota(jnp.int32, (_K * _H, _K * _C), 1)
        e8a = ((q // _H == j // _C) & (q % _H == (j % _C) // 4))
        e8a = e8a.astype(jnp.float32)

        col = lax.broadcasted_iota(jnp.int32, (_C, cm), 1)
        row = lax.broadcasted_iota(jnp.int32, (_C, cm), 0)
        e32 = (col // _M == row).astype(jnp.float32)

        g = g_ref[:]
        d = d_ref[:]
        w = w_ref[:]
        guided = g * jnp.dot(d, e8a, preferred_element_type=jnp.float32)
        acc = None
        for k0 in range(0, _K, 4):
            terms = []
            for k in range(k0, k0 + 4):
                gk = guided[:, k * _C:(k + 1) * _C]
                wk = w[:, k * _M:(k + 1) * _M]
                grep = jnp.dot(gk, e32, preferred_element_type=jnp.float32)
                wrep = jnp.tile(wk, (1, _C))
                terms.append(grep * wrep)
            part = (terms[0] + terms[1]) + (terms[2] + terms[3])
            acc = part if acc is None else acc + part
        o_ref[:] = acc

    return pl.pallas_call(
        body,
        grid=(n // P,),
        in_specs=[
            pl.BlockSpec((P, _K * _C), lambda i: (i, 0)),
            pl.BlockSpec((P, _K * _H), lambda i: (i, 0)),
            pl.BlockSpec((P, _K * _M), lambda i: (i, 0)),
        ],
        out_specs=pl.BlockSpec((P, cm), lambda i: (i, 0)),
        out_shape=jax.ShapeDtypeStruct((n, cm), jnp.float32),
    )(gathered2, guid2, w2)


def kernel(input_features, neighbor_inds, guidance, weightnet):
    b, n, c = input_features.shape
    k = neighbor_inds.shape[2]
    h = guidance.shape[3]
    m = weightnet.shape[3]

    table = input_features.reshape(n, c)
    idx = neighbor_inds.reshape(n * k).astype(jnp.int32)
    gathered = _sc_gather(table, idx)

    out = _tc_contract(
        gathered.reshape(n, k * c),
        guidance.reshape(n, k * h),
        weightnet.reshape(n, k * m),
    )
    return out.reshape(b, n, c * m)
